# Initial kernel scaffold; baseline (speedup 1.0000x reference)
import jax
import jax.numpy as jnp
from jax.experimental import pallas as pl


def kernel(param, values, indices):
    raise NotImplementedError("write your pallas kernel here")


# speedup vs baseline: 5.3602x; 5.3602x over previous
import jax
import jax.numpy as jnp
from jax import lax
from jax.experimental import pallas as pl
from jax.experimental.pallas import tpu as pltpu
from jax.experimental.pallas import tpu_sc as plsc

_NUMEL = 16777216
_NNZ = 1677721
_LR = 0.1

_NC = 2
_NS = 16
_NW = _NC * _NS
_B = 128
_ROWS = 416
_J = 8
_GROUPS = _ROWS // _J
_P = _ROWS * _B
_TOTAL = _NW * _P
_NROWS = _TOTAL // _B


def _sc_body(idx_hbm, val_hbm, param_hbm, out_ref,
             idx_v, val_v, gat_v, sem_ld, sem_g, sem_s):
    c = lax.axis_index("c")
    s = lax.axis_index("s")
    wid = s * _NC + c
    row0 = wid * _ROWS

    @pl.loop(0, _GROUPS)
    def _grp(g):
        r = row0 + g * _J
        ld_i = pltpu.make_async_copy(idx_hbm.at[pl.ds(r, _J)], idx_v, sem_ld)
        ld_v = pltpu.make_async_copy(val_hbm.at[pl.ds(r, _J)], val_v, sem_ld)
        ld_i.start()
        ld_v.start()
        ld_i.wait()
        ld_v.wait()
        gats = [
            pltpu.make_async_copy(param_hbm.at[idx_v.at[j]], gat_v.at[j], sem_g)
            for j in range(_J)
        ]
        for cp in gats:
            cp.start()
        for cp in gats:
            cp.wait()
        for j in range(_J):
            for i in range(_B // 16):
                sl = pl.ds(i * 16, 16)
                gat_v[j, sl] = gat_v[j, sl] + _LR * val_v[j, sl]
        scs = [
            pltpu.make_async_copy(gat_v.at[j], out_ref.at[idx_v.at[j]], sem_s)
            for j in range(_J)
        ]
        for cp in scs:
            cp.start()
        for cp in scs:
            cp.wait()


_sc_update = pl.kernel(
    _sc_body,
    out_type=(),
    mesh=plsc.VectorSubcoreMesh(core_axis_name="c", subcore_axis_name="s"),
    scratch_types=[
        pltpu.VMEM((_J, _B), jnp.int32),
        pltpu.VMEM((_J, _B), jnp.float32),
        pltpu.VMEM((_J, _B), jnp.float32),
        pltpu.SemaphoreType.DMA,
        pltpu.SemaphoreType.DMA,
        pltpu.SemaphoreType.DMA,
    ],
)


def kernel(param, values, indices):
    idx = indices.astype(jnp.int32)
    pad = _TOTAL - _NNZ
    idxp = jnp.pad(idx, (0, pad), mode="wrap").reshape(_NROWS, _B)
    valp = jnp.pad(values, (0, pad), mode="wrap").reshape(_NROWS, _B)
    out_ref = jax.new_ref(param)
    _sc_update(idxp, valp, param, out_ref)
    return out_ref[...]
